# baseline (speedup 1.0000x reference)
import functools

import jax
import jax.numpy as jnp
from jax import lax
from jax.experimental import pallas as pl
from jax.experimental.pallas import tpu as pltpu
from jax.experimental.pallas import tpu_sc as plsc

N = 100000
D1 = 128
D2 = 64
C = 8
B = 100
NSEG = B * C

NC = 2
NS = 16
L = 16
NW = NC * NS

CH_A = 64
NFULL_A = N // CH_A
TAIL_A = N - NFULL_A * CH_A
PER_A = NFULL_A // NW
EXTRA_A = NFULL_A - PER_A * NW
MAXC_A = PER_A + 1
IDS_MAX = MAXC_A * CH_A
XW = CH_A * D1

CHUNK = 128
NFULL = N // CHUNK
TAIL = N - NFULL * CHUNK
PER = NFULL // NW
EXTRA = NFULL - PER * NW
MAXC = PER + 1


def _make_mesh():
    return plsc.VectorSubcoreMesh(
        core_axis_name="c", subcore_axis_name="s", num_cores=NC, num_subcores=NS
    )


def _stage_a_kernel():
    return pl.kernel(
        _stage_a,
        out_type=(
            jax.ShapeDtypeStruct((NW, NSEG * D1), jnp.float32),
            jax.ShapeDtypeStruct((NW, NSEG), jnp.float32),
        ),
        mesh=_make_mesh(),
        scratch_types=[
            pltpu.VMEM(((2 * CH_A + TAIL_A) * D1,), jnp.float32),
            pltpu.VMEM((IDS_MAX,), jnp.int32),
            pltpu.VMEM((IDS_MAX,), jnp.int32),
            pltpu.VMEM((NSEG,), jnp.float32),
            pltpu.VMEM((NSEG * D1,), jnp.float32),
            pltpu.SemaphoreType.DMA,
            pltpu.SemaphoreType.DMA,
        ],
        compiler_params=pltpu.CompilerParams(needs_layout_passes=False),
    )


def _stage_a(x_hbm, b_hbm, c_hbm, zsum_hbm, zcnt_hbm,
             psum_hbm, pcnt_hbm,
             xbuf, ball, call, cnt_local, acc, sem_in, sem_z):
    cid = lax.axis_index("c")
    sid = lax.axis_index("s")
    wid = cid * NS + sid
    start = wid * PER_A + jnp.minimum(wid, EXTRA_A)
    count = PER_A + jnp.where(wid < EXTRA_A, 1, 0)

    pltpu.async_copy(x_hbm.at[pl.ds(start * XW, XW)],
                     xbuf.at[pl.ds(0, XW)], sem_in)
    pltpu.async_copy(x_hbm.at[pl.ds((start + 1) * XW, XW)],
                     xbuf.at[pl.ds(XW, XW)], sem_in)
    pltpu.async_copy(zsum_hbm, acc, sem_z)

    pltpu.sync_copy(zcnt_hbm, cnt_local)

    @pl.when(count == MAXC_A)
    def _():
        pltpu.sync_copy(b_hbm.at[pl.ds(start * CH_A, MAXC_A * CH_A)],
                        ball.at[pl.ds(0, MAXC_A * CH_A)])
        pltpu.sync_copy(c_hbm.at[pl.ds(start * CH_A, MAXC_A * CH_A)],
                        call.at[pl.ds(0, MAXC_A * CH_A)])

    @pl.when(jnp.logical_and(count == PER_A, wid < NW - 1))
    def _():
        pltpu.sync_copy(b_hbm.at[pl.ds(start * CH_A, PER_A * CH_A)],
                        ball.at[pl.ds(0, PER_A * CH_A)])
        pltpu.sync_copy(c_hbm.at[pl.ds(start * CH_A, PER_A * CH_A)],
                        call.at[pl.ds(0, PER_A * CH_A)])

    @pl.when(wid == NW - 1)
    def _():
        pltpu.sync_copy(b_hbm.at[pl.ds(start * CH_A, PER_A * CH_A + TAIL_A)],
                        ball.at[pl.ds(0, PER_A * CH_A + TAIL_A)])
        pltpu.sync_copy(c_hbm.at[pl.ds(start * CH_A, PER_A * CH_A + TAIL_A)],
                        call.at[pl.ds(0, PER_A * CH_A + TAIL_A)])

    pltpu.make_async_copy(zsum_hbm, acc, sem_z).wait()

    ones16 = jnp.ones((L,), jnp.float32)
    iota16 = lax.iota(jnp.int32, L)

    def chunk_body(j, carry):
        bufoff = jnp.bitwise_and(j, 1) * XW
        pltpu.make_async_copy(x_hbm.at[pl.ds(0, XW)],
                             xbuf.at[pl.ds(0, XW)], sem_in).wait()

        def group_body(g, carry2):
            off = j * CH_A + g * L
            keyv = ball[pl.ds(off, L)] * C + call[pl.ds(off, L)]
            plsc.addupdate_scatter(cnt_local, [keyv], ones16)
            kbv = keyv * D1
            xg = bufoff + g * (L * D1)
            for r in range(L):
                kb = kbv[r]
                for i in range(D1 // L):
                    idx = iota16 + (kb + i * L)
                    val = xbuf[pl.ds(xg + r * D1 + i * L, L)]
                    plsc.addupdate_scatter(acc, [idx], val)
            return carry2

        lax.fori_loop(0, CH_A // L, group_body, 0, unroll=False)

        @pl.when(j + 2 < count)
        def _():
            nb = (start + j + 2) * XW
            pltpu.async_copy(x_hbm.at[pl.ds(nb, XW)],
                             xbuf.at[pl.ds(bufoff, XW)], sem_in)
        return carry

    lax.fori_loop(0, count, chunk_body, 0, unroll=False)

    @pl.when(wid == NW - 1)
    def _():
        tb = NFULL_A * XW
        toff = 2 * XW
        pltpu.sync_copy(x_hbm.at[pl.ds(tb, TAIL_A * D1)],
                        xbuf.at[pl.ds(toff, TAIL_A * D1)])

        def tgroup_body(g, carry2):
            off = PER_A * CH_A + g * L
            keyv = ball[pl.ds(off, L)] * C + call[pl.ds(off, L)]
            plsc.addupdate_scatter(cnt_local, [keyv], ones16)
            kbv = keyv * D1
            xg = toff + g * (L * D1)
            for r in range(L):
                kb = kbv[r]
                for i in range(D1 // L):
                    idx = iota16 + (kb + i * L)
                    val = xbuf[pl.ds(xg + r * D1 + i * L, L)]
                    plsc.addupdate_scatter(acc, [idx], val)
            return carry2

        lax.fori_loop(0, TAIL_A // L, tgroup_body, 0, unroll=False)

    pltpu.sync_copy(cnt_local, pcnt_hbm.at[wid])
    pltpu.sync_copy(acc, psum_hbm.at[wid])


def _mid_body(ps_ref, pc_ref, rm_ref, w1b_ref, b1b_ref,
              w2b_ref, b2_ref, out_ref):
    hi = jax.lax.Precision.HIGHEST
    counts2 = jnp.sum(pc_ref[...], axis=0)
    denom = jnp.sum(counts2 * counts2, axis=1, keepdims=True)
    denom = jnp.where(denom > 0.0, denom, 1.0)
    ratio2 = counts2 / denom
    rexp = jnp.dot(ratio2, rm_ref[...], precision=hi)
    cs = jnp.sum(ps_ref[...], axis=0)
    r2 = cs * rexp
    h2 = jnp.dot(r2, w1b_ref[...], precision=hi) + b1b_ref[...]
    h2 = jnp.where(h2 >= 0.0, h2, 0.45 * h2)
    s2 = jnp.dot(h2, w2b_ref[...], precision=hi) + b2_ref[...]
    masked = jnp.where(counts2 > 0.0, s2, -1e30)
    smax = jnp.max(masked, axis=1, keepdims=True)
    smax = jnp.where(smax > -1e29, smax, 0.0)
    e2 = jnp.exp(s2 - smax)
    ssum = jnp.sum(counts2 * e2, axis=1, keepdims=True)
    out_ref[...] = e2 / (ssum + 1e-16)


_stage_b = pl.pallas_call(
    _mid_body,
    out_shape=jax.ShapeDtypeStruct((B, C), jnp.float32),
)


def _stage_c_kernel():
    return pl.kernel(
        _stage_c,
        out_type=jax.ShapeDtypeStruct((N,), jnp.float32),
        mesh=_make_mesh(),
        scratch_types=[
            pltpu.VMEM((NSEG,), jnp.float32),
            pltpu.VMEM((MAXC * CHUNK,), jnp.int32),
            pltpu.VMEM((MAXC * CHUNK,), jnp.int32),
            pltpu.VMEM((CHUNK,), jnp.float32),
            pltpu.VMEM((TAIL,), jnp.int32),
            pltpu.VMEM((TAIL,), jnp.int32),
            pltpu.VMEM((TAIL,), jnp.float32),
        ],
        compiler_params=pltpu.CompilerParams(needs_layout_passes=False),
    )


def _stage_c(w_hbm, b_hbm, c_hbm, out_hbm,
             wbuf, ball, call, obuf, bbuf_t, cbuf_t, obuf_t):
    cid = lax.axis_index("c")
    sid = lax.axis_index("s")
    wid = cid * NS + sid
    start = wid * PER + jnp.minimum(wid, EXTRA)
    count = PER + jnp.where(wid < EXTRA, 1, 0)

    pltpu.sync_copy(w_hbm, wbuf)

    @pl.when(count == PER + 1)
    def _():
        pltpu.sync_copy(b_hbm.at[pl.ds(start * CHUNK, MAXC * CHUNK)],
                        ball.at[pl.ds(0, MAXC * CHUNK)])
        pltpu.sync_copy(c_hbm.at[pl.ds(start * CHUNK, MAXC * CHUNK)],
                        call.at[pl.ds(0, MAXC * CHUNK)])

    @pl.when(count == PER)
    def _():
        pltpu.sync_copy(b_hbm.at[pl.ds(start * CHUNK, PER * CHUNK)],
                        ball.at[pl.ds(0, PER * CHUNK)])
        pltpu.sync_copy(c_hbm.at[pl.ds(start * CHUNK, PER * CHUNK)],
                        call.at[pl.ds(0, PER * CHUNK)])

    def chunk_body(j, carry):
        base = (start + j) * CHUNK
        for i in range(CHUNK // L):
            off = j * CHUNK + i * L
            key = ball[pl.ds(off, L)] * C + call[pl.ds(off, L)]
            obuf[pl.ds(i * L, L)] = plsc.load_gather(wbuf, [key])
        pltpu.sync_copy(obuf, out_hbm.at[pl.ds(base, CHUNK)])
        return carry

    lax.fori_loop(0, count, chunk_body, 0, unroll=False)

    @pl.when(wid == NW - 1)
    def _():
        tbase = NFULL * CHUNK
        pltpu.sync_copy(b_hbm.at[pl.ds(tbase, TAIL)], bbuf_t)
        pltpu.sync_copy(c_hbm.at[pl.ds(tbase, TAIL)], cbuf_t)
        for i in range(TAIL // L):
            key = bbuf_t[pl.ds(i * L, L)] * C + cbuf_t[pl.ds(i * L, L)]
            obuf_t[pl.ds(i * L, L)] = plsc.load_gather(wbuf, [key])
        pltpu.sync_copy(obuf_t, out_hbm.at[pl.ds(tbase, TAIL)])


def kernel(x, cls, batch, W1, b1, W2, b2):
    cls_i = cls.astype(jnp.int32)
    batch_i = batch.astype(jnp.int32)

    x_flat = x.reshape(N * D1)
    zsum = jnp.zeros((NSEG * D1,), jnp.float32)
    zcnt = jnp.zeros((NSEG,), jnp.float32)

    psum, pcnt = _stage_a_kernel()(x_flat, batch_i, cls_i, zsum, zcnt)

    ps = psum.reshape(NW, B, C * D1)
    pc = pcnt.reshape(NW, B, C)

    eye = jnp.eye(C, dtype=jnp.float32)
    rm = jnp.kron(eye, jnp.ones((1, D1), jnp.float32))
    w1b = jnp.kron(eye, W1.T)
    b1b = jnp.tile(b1, C).reshape(1, C * D2)
    w2b = jnp.kron(eye, W2.T)
    b2b = b2.reshape(1, 1)

    w2 = _stage_b(ps, pc, rm, w1b, b1b, w2b, b2b)
    wseg = w2.reshape(NSEG)

    out = _stage_c_kernel()(wseg, batch_i, cls_i)
    return out.reshape(N, 1)


# speedup vs baseline: 2.2867x; 2.2867x over previous
import functools

import jax
import jax.numpy as jnp
from jax import lax
from jax.experimental import pallas as pl
from jax.experimental.pallas import tpu as pltpu
from jax.experimental.pallas import tpu_sc as plsc

N = 100000
D1 = 128
D2 = 64
C = 8
B = 100
NSEG = B * C

NC = 2
NS = 16
L = 16
NW = NC * NS

CHUNK = 128
NFULL = N // CHUNK
TAIL = N - NFULL * CHUNK
PER = NFULL // NW
EXTRA = NFULL - PER * NW
MAXC = PER + 1

def _make_mesh():
    return plsc.VectorSubcoreMesh(
        core_axis_name="c", subcore_axis_name="s", num_cores=NC, num_subcores=NS
    )


def _wid_info(wid):
    start = wid * PER + jnp.minimum(wid, EXTRA)
    count = PER + jnp.where(wid < EXTRA, 1, 0)
    return start, count


def _stage_a_kernel():
    return pl.kernel(
        _stage_a,
        out_type=(
            jax.ShapeDtypeStruct((NC, NSEG, D1), jnp.float32),
            jax.ShapeDtypeStruct((NW, NSEG), jnp.float32),
        ),
        mesh=_make_mesh(),
        scratch_types=[
            pltpu.VMEM((3, CHUNK, D1), jnp.float32),
            pltpu.VMEM((MAXC * CHUNK,), jnp.int32),
            pltpu.VMEM((MAXC * CHUNK,), jnp.int32),
            pltpu.VMEM((3, CHUNK), jnp.int32),
            pltpu.VMEM((NSEG,), jnp.float32),
            pltpu.VMEM((TAIL, D1), jnp.float32),
            pltpu.VMEM((TAIL,), jnp.int32),
            pltpu.VMEM((TAIL,), jnp.int32),
            pltpu.VMEM((TAIL,), jnp.int32),
            pltpu.VMEM_SHARED((NSEG, D1), jnp.float32),
            pltpu.SemaphoreType.DMA,
            pltpu.SemaphoreType.DMA,
        ],
        compiler_params=pltpu.CompilerParams(needs_layout_passes=False),
    )


def _stage_a(x_hbm, b_hbm, c_hbm, zsum_hbm, zcnt_hbm,
             psum_hbm, pcnt_hbm,
             xbuf2, ball, call, kbuf2, cnt_local,
             xbuf_t, bbuf_t, cbuf_t, kbuf_t,
             acc_sum, sem_in, sem_sc):
    cid = lax.axis_index("c")
    sid = lax.axis_index("s")
    wid = cid * NS + sid
    start, count = _wid_info(wid)

    pltpu.async_copy(x_hbm.at[pl.ds(start * CHUNK, CHUNK), :], xbuf2.at[0],
                     sem_in)
    pltpu.async_copy(x_hbm.at[pl.ds((start + 1) * CHUNK, CHUNK), :],
                     xbuf2.at[1], sem_in)

    @pl.when(sid == 0)
    def _():
        pltpu.sync_copy(zsum_hbm, acc_sum)

    pltpu.sync_copy(zcnt_hbm, cnt_local)

    @pl.when(count == PER + 1)
    def _():
        pltpu.sync_copy(b_hbm.at[pl.ds(start * CHUNK, MAXC * CHUNK)],
                        ball.at[pl.ds(0, MAXC * CHUNK)])
        pltpu.sync_copy(c_hbm.at[pl.ds(start * CHUNK, MAXC * CHUNK)],
                        call.at[pl.ds(0, MAXC * CHUNK)])

    @pl.when(count == PER)
    def _():
        pltpu.sync_copy(b_hbm.at[pl.ds(start * CHUNK, PER * CHUNK)],
                        ball.at[pl.ds(0, PER * CHUNK)])
        pltpu.sync_copy(c_hbm.at[pl.ds(start * CHUNK, PER * CHUNK)],
                        call.at[pl.ds(0, PER * CHUNK)])

    plsc.subcore_barrier()

    ones16 = jnp.ones((L,), jnp.float32)

    def chunk_body(j, carry):
        par = lax.rem(j, 3)
        parn = lax.rem(j + 2, 3)
        pltpu.make_async_copy(x_hbm.at[pl.ds(0, CHUNK), :], xbuf2.at[par],
                              sem_in).wait()

        @pl.when(j > 0)
        def _():
            pltpu.make_async_copy(
                xbuf2.at[parn], acc_sum.at[kbuf2.at[parn]], sem_sc
            ).wait()

        @pl.when(j + 2 < count)
        def _():
            nbase = (start + j + 2) * CHUNK
            pltpu.async_copy(x_hbm.at[pl.ds(nbase, CHUNK), :],
                             xbuf2.at[parn], sem_in)

        kbuf = kbuf2.at[par]
        for i in range(CHUNK // L):
            off = j * CHUNK + i * L
            key = ball[pl.ds(off, L)] * C + call[pl.ds(off, L)]
            kbuf[pl.ds(i * L, L)] = key
            plsc.addupdate_scatter(cnt_local, [key], ones16)
        pltpu.async_copy(xbuf2.at[par], acc_sum.at[kbuf], sem_sc, add=True)
        return carry

    lax.fori_loop(0, count, chunk_body, 0, unroll=False)

    lastp = lax.rem(count - 1, 3)
    pltpu.make_async_copy(
        xbuf2.at[lastp], acc_sum.at[kbuf2.at[lastp]], sem_sc
    ).wait()

    @pl.when(wid == NW - 1)
    def _():
        tbase = NFULL * CHUNK
        pltpu.sync_copy(b_hbm.at[pl.ds(tbase, TAIL)], bbuf_t)
        pltpu.sync_copy(c_hbm.at[pl.ds(tbase, TAIL)], cbuf_t)
        pltpu.sync_copy(x_hbm.at[pl.ds(tbase, TAIL), :], xbuf_t)
        for i in range(TAIL // L):
            key = bbuf_t[pl.ds(i * L, L)] * C + cbuf_t[pl.ds(i * L, L)]
            kbuf_t[pl.ds(i * L, L)] = key
            plsc.addupdate_scatter(cnt_local, [key], ones16)
        pltpu.sync_copy(xbuf_t, acc_sum.at[kbuf_t], add=True)

    pltpu.sync_copy(cnt_local, pcnt_hbm.at[wid])

    plsc.subcore_barrier()

    @pl.when(sid == 0)
    def _():
        pltpu.sync_copy(acc_sum, psum_hbm.at[cid])


def _mid_body(cs0_ref, cs1_ref, pc_ref, rm_ref, w1b_ref, b1b_ref,
              w2b_ref, b2_ref, out_ref):
    hi = jax.lax.Precision.HIGHEST
    counts2 = jnp.sum(pc_ref[...], axis=0)
    denom = jnp.sum(counts2 * counts2, axis=1, keepdims=True)
    denom = jnp.where(denom > 0.0, denom, 1.0)
    ratio2 = counts2 / denom
    rexp = jnp.dot(ratio2, rm_ref[...], precision=hi)
    r2 = (cs0_ref[...] + cs1_ref[...]) * rexp
    h2 = jnp.dot(r2, w1b_ref[...], precision=hi) + b1b_ref[...]
    h2 = jnp.where(h2 >= 0.0, h2, 0.45 * h2)
    s2 = jnp.dot(h2, w2b_ref[...], precision=hi) + b2_ref[...]
    masked = jnp.where(counts2 > 0.0, s2, -1e30)
    smax = jnp.max(masked, axis=1, keepdims=True)
    smax = jnp.where(smax > -1e29, smax, 0.0)
    e2 = jnp.exp(s2 - smax)
    ssum = jnp.sum(counts2 * e2, axis=1, keepdims=True)
    out_ref[...] = e2 / (ssum + 1e-16)


_stage_b = pl.pallas_call(
    _mid_body,
    out_shape=jax.ShapeDtypeStruct((B, C), jnp.float32),
)


def _stage_c_kernel():
    return pl.kernel(
        _stage_c,
        out_type=jax.ShapeDtypeStruct((N,), jnp.float32),
        mesh=_make_mesh(),
        scratch_types=[
            pltpu.VMEM((NSEG,), jnp.float32),
            pltpu.VMEM((MAXC * CHUNK,), jnp.int32),
            pltpu.VMEM((MAXC * CHUNK,), jnp.int32),
            pltpu.VMEM((2, CHUNK), jnp.float32),
            pltpu.VMEM((TAIL,), jnp.int32),
            pltpu.VMEM((TAIL,), jnp.int32),
            pltpu.VMEM((TAIL,), jnp.float32),
            pltpu.SemaphoreType.DMA,
            pltpu.SemaphoreType.DMA,
        ],
        compiler_params=pltpu.CompilerParams(needs_layout_passes=False),
    )


def _stage_c(w_hbm, b_hbm, c_hbm, out_hbm,
             wbuf, ball, call, obuf2, bbuf_t, cbuf_t, obuf_t,
             sem_w, sem_out):
    cid = lax.axis_index("c")
    sid = lax.axis_index("s")
    wid = cid * NS + sid
    start, count = _wid_info(wid)

    pltpu.async_copy(w_hbm, wbuf, sem_w)

    @pl.when(count == PER + 1)
    def _():
        pltpu.sync_copy(b_hbm.at[pl.ds(start * CHUNK, MAXC * CHUNK)],
                        ball.at[pl.ds(0, MAXC * CHUNK)])
        pltpu.sync_copy(c_hbm.at[pl.ds(start * CHUNK, MAXC * CHUNK)],
                        call.at[pl.ds(0, MAXC * CHUNK)])

    @pl.when(count == PER)
    def _():
        pltpu.sync_copy(b_hbm.at[pl.ds(start * CHUNK, PER * CHUNK)],
                        ball.at[pl.ds(0, PER * CHUNK)])
        pltpu.sync_copy(c_hbm.at[pl.ds(start * CHUNK, PER * CHUNK)],
                        call.at[pl.ds(0, PER * CHUNK)])

    pltpu.make_async_copy(w_hbm, wbuf, sem_w).wait()

    def chunk_body(j, carry):
        par = jnp.bitwise_and(j, 1)
        base = (start + j) * CHUNK

        @pl.when(j > 1)
        def _():
            pltpu.make_async_copy(obuf2.at[par], out_hbm.at[pl.ds(0, CHUNK)],
                                  sem_out).wait()

        ob = obuf2.at[par]
        for i in range(CHUNK // L):
            off = j * CHUNK + i * L
            key = ball[pl.ds(off, L)] * C + call[pl.ds(off, L)]
            ob[pl.ds(i * L, L)] = plsc.load_gather(wbuf, [key])
        pltpu.async_copy(obuf2.at[par], out_hbm.at[pl.ds(base, CHUNK)],
                         sem_out)
        return carry

    lax.fori_loop(0, count, chunk_body, 0, unroll=False)

    pltpu.make_async_copy(obuf2.at[0], out_hbm.at[pl.ds(0, CHUNK)],
                          sem_out).wait()
    pltpu.make_async_copy(obuf2.at[0], out_hbm.at[pl.ds(0, CHUNK)],
                          sem_out).wait()

    @pl.when(wid == NW - 1)
    def _():
        tbase = NFULL * CHUNK
        pltpu.sync_copy(b_hbm.at[pl.ds(tbase, TAIL)], bbuf_t)
        pltpu.sync_copy(c_hbm.at[pl.ds(tbase, TAIL)], cbuf_t)
        for i in range(TAIL // L):
            key = bbuf_t[pl.ds(i * L, L)] * C + cbuf_t[pl.ds(i * L, L)]
            obuf_t[pl.ds(i * L, L)] = plsc.load_gather(wbuf, [key])
        pltpu.sync_copy(obuf_t, out_hbm.at[pl.ds(tbase, TAIL)])


def kernel(x, cls, batch, W1, b1, W2, b2):
    cls_i = cls.astype(jnp.int32)
    batch_i = batch.astype(jnp.int32)

    zsum = jnp.zeros((NSEG, D1), jnp.float32)
    zcnt = jnp.zeros((NSEG,), jnp.float32)

    psum, pcnt = _stage_a_kernel()(x, batch_i, cls_i, zsum, zcnt)

    cs0 = psum[0].reshape(B, C * D1)
    cs1 = psum[1].reshape(B, C * D1)
    pc = pcnt.reshape(NW, B, C)

    eye = jnp.eye(C, dtype=jnp.float32)
    rm = jnp.kron(eye, jnp.ones((1, D1), jnp.float32))
    w1b = jnp.kron(eye, W1.T)
    b1b = jnp.tile(b1, C).reshape(1, C * D2)
    w2b = jnp.kron(eye, W2.T)
    b2b = b2.reshape(1, 1)

    w2 = _stage_b(cs0, cs1, pc, rm, w1b, b1b, w2b, b2b)
    wseg = w2.reshape(NSEG)

    out = _stage_c_kernel()(wseg, batch_i, cls_i)
    return out.reshape(N, 1)
